# baseline (speedup 1.0000x reference)
import math

import jax
import jax.numpy as jnp
from jax import lax
from jax.experimental import pallas as pl
from jax.experimental.pallas import tpu as pltpu
from jax.experimental.pallas import tpu_sc as plsc

N = 10000
E = 320000
C = 128
H = 8
DH = 16
SCALE = 1.0 / math.sqrt(DH)

NC = 2
NS = 16
NW = NC * NS
EW = E // NW

SCH = 80
SNCH = EW // SCH
SNGR = SCH // 16
IBLK = 5
NBLK = SNCH // IBLK

RPT = 624


def _qkv_body(f_ref, w_ref, b_ref, oq_ref, ok_ref, ov_ref):
    acc = jnp.dot(f_ref[...], w_ref[...], preferred_element_type=jnp.float32)
    acc = acc + b_ref[...]
    oq_ref[...] = acc[:, 0:C]
    ok_ref[...] = acc[:, C:2 * C]
    ov_ref[...] = acc[:, 2 * C:3 * C]


def _qkv(features, wcat, bcat):
    bn = 1000
    return pl.pallas_call(
        _qkv_body,
        grid=(N // bn,),
        in_specs=[
            pl.BlockSpec((bn, C), lambda i: (i, 0)),
            pl.BlockSpec((C, 3 * C), lambda i: (0, 0)),
            pl.BlockSpec((1, 3 * C), lambda i: (0, 0)),
        ],
        out_specs=[
            pl.BlockSpec((bn, C), lambda i: (i, 0)),
            pl.BlockSpec((bn, C), lambda i: (i, 0)),
            pl.BlockSpec((bn, C), lambda i: (i, 0)),
        ],
        out_shape=[jax.ShapeDtypeStruct((N, C), jnp.float32)] * 3,
    )(features, wcat, bcat)


LDEPTH = 4


def _logits_body(row4, col4, q_h, k_h, out_h,
                 ribuf, cibuf,
                 qr0, qr1, qr2, qr3, kr0, kr1, kr2, kr3, lgb,
                 gsem0, gsem1, gsem2, gsem3,
                 osem0, osem1, osem2, osem3):
    cid = lax.axis_index("c")
    sid = lax.axis_index("s")
    wid = sid * NC + cid
    base = wid * EW
    qr = [qr0, qr1, qr2, qr3]
    kr = [kr0, kr1, kr2, kr3]
    gsem = [gsem0, gsem1, gsem2, gsem3]
    osem = [osem0, osem1, osem2, osem3]

    def prefetch(c2, b2):
        @pl.when(c2 <= SNCH - 1)
        def _():
            blk = c2 // IBLK

            @pl.when(c2 % IBLK == 0)
            def _():
                pltpu.sync_copy(row4.at[wid, blk], ribuf.at[blk % 2])
                pltpu.sync_copy(col4.at[wid, blk], cibuf.at[blk % 2])

            pltpu.async_copy(q_h.at[ribuf.at[blk % 2, c2 % IBLK]],
                             qr[b2], gsem[b2])
            pltpu.async_copy(k_h.at[cibuf.at[blk % 2, c2 % IBLK]],
                             kr[b2], gsem[b2])

    def process(c, b):
        pltpu.make_async_copy(q_h.at[pl.ds(0, SCH)], qr[b], gsem[b]).wait()
        pltpu.make_async_copy(k_h.at[pl.ds(0, SCH)], kr[b], gsem[b]).wait()

        @pl.when(c >= LDEPTH)
        def _():
            pltpu.make_async_copy(out_h.at[pl.ds(0, H * SCH)],
                                  lgb.at[pl.ds(b * H * SCH, H * SCH)],
                                  osem[b]).wait()

        def group(g, carry2):
            eidx = lax.iota(jnp.int32, 16) + g * 16
            for h in range(H):
                acc = jnp.zeros((16,), jnp.float32)
                for d in range(DH):
                    cc = jnp.full((16,), h * DH + d, jnp.int32)
                    qv = plsc.load_gather(qr[b], [eidx, cc])
                    kv = plsc.load_gather(kr[b], [eidx, cc])
                    acc = acc + qv * kv

                lgb[pl.ds(b * H * SCH + h * SCH + g * 16, 16)] = acc * SCALE
            return carry2

        lax.fori_loop(0, SNGR, group, 0)
        for h in range(H):
            pltpu.async_copy(lgb.at[pl.ds(b * H * SCH + h * SCH, SCH)],
                             out_h.at[pl.ds(h * E + base + c * SCH, SCH)],
                             osem[b])

    for c0 in range(LDEPTH):
        prefetch(jnp.int32(c0), c0)

    def body(i, carry):
        for off in range(LDEPTH):
            c = i * LDEPTH + off
            process(c, off)
            prefetch(c + LDEPTH, off)
        return carry

    lax.fori_loop(0, (SNCH - 1) // LDEPTH, body, 0)
    process(jnp.int32(SNCH - 1), 0)
    for b in range(LDEPTH):
        pltpu.make_async_copy(out_h.at[pl.ds(0, H * SCH)],
                              lgb.at[pl.ds(b * H * SCH, H * SCH)],
                              osem[b]).wait()


def _logits(row4, col4, q, k):
    mesh = plsc.VectorSubcoreMesh(core_axis_name="c", subcore_axis_name="s")
    f = pl.kernel(
        _logits_body,
        out_type=jax.ShapeDtypeStruct((H * E,), jnp.float32),
        mesh=mesh,
        compiler_params=pltpu.CompilerParams(needs_layout_passes=False),
        scratch_types=[
            pltpu.VMEM((2, IBLK, SCH), jnp.int32),
            pltpu.VMEM((2, IBLK, SCH), jnp.int32),
        ] + [pltpu.VMEM((SCH, C), jnp.float32)] * 8 + [
            pltpu.VMEM((LDEPTH * H * SCH,), jnp.float32),
        ] + [pltpu.SemaphoreType.DMA] * 8,
    )
    return f(row4, col4, q, k)


def _stats_body(l_ref, m_ref, z_ref, m_s, z_s):
    i = pl.program_id(0)

    @pl.when(i == 0)
    def _():
        m_s[...] = jnp.full((H, 128), -jnp.inf, jnp.float32)
        z_s[...] = jnp.zeros((H, 128), jnp.float32)

    blk = l_ref[...].reshape(H, -1, 128)
    bm = blk.max(axis=1)
    m_old = m_s[...]
    m_new = jnp.maximum(m_old, bm)
    z_s[...] = z_s[...] * jnp.exp(m_old - m_new) + jnp.exp(
        blk - m_new[:, None, :]).sum(axis=1)
    m_s[...] = m_new

    @pl.when(i == pl.num_programs(0) - 1)
    def _():
        mf = m_s[...].max(axis=1, keepdims=True)
        zf = (z_s[...] * jnp.exp(m_s[...] - mf)).sum(axis=1, keepdims=True)
        m_ref[...] = jnp.broadcast_to(mf, (H, 128))
        z_ref[...] = jnp.broadcast_to(zf, (H, 128))


def _stats(logits):
    bl = 16000
    return pl.pallas_call(
        _stats_body,
        grid=(E // bl,),
        in_specs=[pl.BlockSpec((H, bl), lambda i: (0, i))],
        out_specs=[
            pl.BlockSpec((H, 128), lambda i: (0, 0)),
            pl.BlockSpec((H, 128), lambda i: (0, 0)),
        ],
        out_shape=[jax.ShapeDtypeStruct((H, 128), jnp.float32)] * 2,
        scratch_shapes=[
            pltpu.VMEM((H, 128), jnp.float32),
            pltpu.VMEM((H, 128), jnp.float32),
        ],
    )(logits)


def _scatter_body(row4, col4, v_h, lg_h, m_h, z_h, zero_h, out_h,
                  ribuf, cibuf, vr0, vr1, vr2, vr3, lgv, msv, zsv, osh,
                  gsem0, gsem1, gsem2, gsem3,
                  ssem0, ssem1, ssem2, ssem3,
                  lsem0, lsem1, lsem2, lsem3):
    cid = lax.axis_index("c")
    sid = lax.axis_index("s")
    wid = sid * NC + cid
    base = wid * EW
    vr = [vr0, vr1, vr2, vr3]
    gsem = [gsem0, gsem1, gsem2, gsem3]
    ssem = [ssem0, ssem1, ssem2, ssem3]
    lsem = [lsem0, lsem1, lsem2, lsem3]

    pltpu.sync_copy(zero_h.at[pl.ds(sid * RPT, RPT)], osh.at[pl.ds(sid * RPT, RPT)])

    @pl.when(sid == NS - 1)
    def _():
        pltpu.sync_copy(zero_h.at[pl.ds(NS * RPT, N - NS * RPT)],
                        osh.at[pl.ds(NS * RPT, N - NS * RPT)])

    def prefetch(c2, b2):
        @pl.when(c2 <= SNCH - 1)
        def _():
            blk = c2 // IBLK

            @pl.when(c2 % IBLK == 0)
            def _():
                pltpu.sync_copy(row4.at[wid, blk], ribuf.at[blk % 2])
                pltpu.sync_copy(col4.at[wid, blk], cibuf.at[blk % 2])

            @pl.when(c2 >= 4)
            def _():
                pltpu.make_async_copy(v_h.at[pl.ds(0, SCH)], vr[b2],
                                      ssem[b2]).wait()

            for h in range(H):
                pltpu.async_copy(
                    lg_h.at[pl.ds(h * E + base + c2 * SCH, SCH)],
                    lgv.at[pl.ds(b2 * H * SCH + h * SCH, SCH)], lsem[b2])
            pltpu.async_copy(v_h.at[cibuf.at[blk % 2, c2 % IBLK]],
                             vr[b2], gsem[b2])

    prefetch(jnp.int32(0), 0)
    prefetch(jnp.int32(1), 1)

    plsc.subcore_barrier()

    pltpu.sync_copy(m_h, msv)
    pltpu.sync_copy(z_h, zsv)
    ms = [msv[pl.ds(h * 16, 16)] for h in range(H)]
    rzs = [1.0 / zsv[pl.ds(h * 16, 16)] for h in range(H)]

    def process(c, b):
        blk = c // IBLK
        pltpu.make_async_copy(v_h.at[pl.ds(0, SCH)], vr[b], gsem[b]).wait()
        pltpu.make_async_copy(lg_h.at[pl.ds(0, H * SCH)],
                              lgv.at[pl.ds(b * H * SCH, H * SCH)],
                              lsem[b]).wait()
        for h in range(H):
            for j in range(SNGR):
                sl = pl.ds(b * H * SCH + h * SCH + j * 16, 16)
                lgv[sl] = jnp.exp(lgv[sl] - ms[h]) * rzs[h]

        def group(g, carry2):
            for e in range(16):
                ei = jnp.full((16,), g * 16 + e, jnp.int32)
                for h in range(H):
                    w = plsc.load_gather(lgv, [ei + (b * H * SCH + h * SCH)])
                    didx = lax.iota(jnp.int32, 16) + h * DH
                    x = plsc.load_gather(vr[b], [ei, didx])
                    plsc.store_scatter(vr[b], [ei, didx], x * w)
            return carry2

        lax.fori_loop(0, SNGR, group, 0)
        pltpu.async_copy(vr[b], osh.at[ribuf.at[blk % 2, c % IBLK]],
                         ssem[b], add=True)

    def body(i, carry):
        for off in range(4):
            c = i * 4 + off
            process(c, off)
            prefetch(c + 2, (off + 2) % 4)
        return carry

    lax.fori_loop(0, (SNCH - 1) // 4, body, 0)
    process(jnp.int32(SNCH - 1), 0)
    for b in range(4):
        pltpu.make_async_copy(v_h.at[pl.ds(0, SCH)], vr[b], ssem[b]).wait()

    plsc.subcore_barrier()
    pltpu.sync_copy(osh.at[pl.ds(sid * RPT, RPT)],
                    out_h.at[cid, pl.ds(sid * RPT, RPT)])

    @pl.when(sid == NS - 1)
    def _():
        pltpu.sync_copy(osh.at[pl.ds(NS * RPT, N - NS * RPT)],
                        out_h.at[cid, pl.ds(NS * RPT, N - NS * RPT)])


def _scatter(row4, col4, v, logits, m, z, zeros):
    mesh = plsc.VectorSubcoreMesh(core_axis_name="c", subcore_axis_name="s")
    f = pl.kernel(
        _scatter_body,
        out_type=jax.ShapeDtypeStruct((NC, N, C), jnp.float32),
        mesh=mesh,
        compiler_params=pltpu.CompilerParams(needs_layout_passes=False),
        scratch_types=[
            pltpu.VMEM((2, IBLK, SCH), jnp.int32),
            pltpu.VMEM((2, IBLK, SCH), jnp.int32),
            pltpu.VMEM((SCH, C), jnp.float32),
            pltpu.VMEM((SCH, C), jnp.float32),
            pltpu.VMEM((SCH, C), jnp.float32),
            pltpu.VMEM((SCH, C), jnp.float32),
            pltpu.VMEM((4 * H * SCH,), jnp.float32),
            pltpu.VMEM((H * 16,), jnp.float32),
            pltpu.VMEM((H * 16,), jnp.float32),
            pltpu.VMEM_SHARED((N, C), jnp.float32),
        ] + [pltpu.SemaphoreType.DMA] * 12,
    )
    return f(row4, col4, v, logits, m, z, zeros)


def _out_body(p0_ref, p1_ref, w_ref, b_ref, o_ref):
    o_ref[...] = jnp.dot(p0_ref[0] + p1_ref[0], w_ref[...],
                         preferred_element_type=jnp.float32) + b_ref[...]


def _outproj(partials, wo_t, bo):
    bn = 1000
    return pl.pallas_call(
        _out_body,
        grid=(N // bn,),
        in_specs=[
            pl.BlockSpec((1, bn, C), lambda i: (0, i, 0)),
            pl.BlockSpec((1, bn, C), lambda i: (1, i, 0)),
            pl.BlockSpec((C, C), lambda i: (0, 0)),
            pl.BlockSpec((1, C), lambda i: (0, 0)),
        ],
        out_specs=pl.BlockSpec((bn, C), lambda i: (i, 0)),
        out_shape=jax.ShapeDtypeStruct((N, C), jnp.float32),
    )(partials, partials, wo_t, bo)


def kernel(features, edge_index, Wq, bq, Wk, bk, Wv, bv, Wo, bo):
    row4 = edge_index[0].astype(jnp.int32).reshape(NW, NBLK, IBLK, SCH)
    col4 = edge_index[1].astype(jnp.int32).reshape(NW, NBLK, IBLK, SCH)
    wcat = jnp.concatenate([Wq.T, Wk.T, Wv.T], axis=1)
    bcat = jnp.concatenate([bq, bk, bv]).reshape(1, 3 * C)
    q, k, v = _qkv(features, wcat, bcat)
    logits = _logits(row4, col4, q, k)
    m, z = _stats(logits.reshape(H, E))
    m16 = lax.slice(m, (0, 0), (H, 16)).reshape(H * 16)
    z16 = lax.slice(z, (0, 0), (H, 16)).reshape(H * 16)
    zeros = jnp.zeros((N, C), jnp.float32)
    partials = _scatter(row4, col4, v, logits, m16, z16, zeros)
    return _outproj(partials, Wo.T, bo.reshape(1, C))


# speedup vs baseline: 1.4008x; 1.4008x over previous
import math

import jax
import jax.numpy as jnp
from jax import lax
from jax.experimental import pallas as pl
from jax.experimental.pallas import tpu as pltpu
from jax.experimental.pallas import tpu_sc as plsc

N = 10000
E = 320000
C = 128
H = 8
DH = 16
SCALE = 1.0 / math.sqrt(DH)

NC = 2
NS = 16
NW = NC * NS
EW = E // NW

SCH = 80
SNCH = EW // SCH
SNGR = SCH // 16
IBLK = 5
NBLK = SNCH // IBLK

RPT = 624


def _qkv_body(f_ref, w_ref, b_ref, oq_ref, ok_ref, ov_ref):
    acc = jnp.dot(f_ref[...], w_ref[...], preferred_element_type=jnp.float32)
    acc = acc + b_ref[...]
    oq_ref[...] = acc[:, 0:C].astype(jnp.bfloat16)
    ok_ref[...] = acc[:, C:2 * C].astype(jnp.bfloat16)
    ov_ref[...] = acc[:, 2 * C:3 * C]


def _qkv(features, wcat, bcat):
    bn = 1000
    return pl.pallas_call(
        _qkv_body,
        grid=(N // bn,),
        in_specs=[
            pl.BlockSpec((bn, C), lambda i: (i, 0)),
            pl.BlockSpec((C, 3 * C), lambda i: (0, 0)),
            pl.BlockSpec((1, 3 * C), lambda i: (0, 0)),
        ],
        out_specs=[
            pl.BlockSpec((bn, C), lambda i: (i, 0)),
            pl.BlockSpec((bn, C), lambda i: (i, 0)),
            pl.BlockSpec((bn, C), lambda i: (i, 0)),
        ],
        out_shape=[
            jax.ShapeDtypeStruct((N, C), jnp.bfloat16),
            jax.ShapeDtypeStruct((N, C), jnp.bfloat16),
            jax.ShapeDtypeStruct((N, C), jnp.float32),
        ],
    )(features, wcat, bcat)


LDEPTH = 2


def _logits_body(row4, col4, q_h, k_h, out_h,
                 ribuf, cibuf,
                 qr0, qr1, kr0, kr1, lgb,
                 gsem0, gsem1,
                 osem0, osem1):
    cid = lax.axis_index("c")
    sid = lax.axis_index("s")
    wid = sid * NC + cid
    base = wid * EW
    qr = [qr0, qr1]
    kr = [kr0, kr1]
    gsem = [gsem0, gsem1]
    osem = [osem0, osem1]

    def prefetch(c2, b2):
        @pl.when(c2 <= SNCH - 1)
        def _():
            blk = c2 // IBLK

            @pl.when(c2 % IBLK == 0)
            def _():
                pltpu.sync_copy(row4.at[wid, blk], ribuf.at[blk % 2])
                pltpu.sync_copy(col4.at[wid, blk], cibuf.at[blk % 2])

            pltpu.async_copy(q_h.at[ribuf.at[blk % 2, c2 % IBLK]],
                             qr[b2], gsem[b2])
            pltpu.async_copy(k_h.at[cibuf.at[blk % 2, c2 % IBLK]],
                             kr[b2], gsem[b2])

    def process(c, b):
        pltpu.make_async_copy(q_h.at[pl.ds(0, SCH)], qr[b], gsem[b]).wait()
        pltpu.make_async_copy(q_h.at[pl.ds(0, SCH)], kr[b], gsem[b]).wait()

        @pl.when(c >= LDEPTH)
        def _():
            pltpu.make_async_copy(out_h.at[pl.ds(0, H * SCH)],
                                  lgb.at[pl.ds(b * H * SCH, H * SCH)],
                                  osem[b]).wait()

        def group(g, carry2):
            eidx = lax.iota(jnp.int32, 16) + g * 16
            for h in range(H):
                acc = jnp.zeros((16,), jnp.float32)
                for w in range(DH // 2):
                    cc = jnp.full((16,), h * (DH // 2) + w, jnp.int32)
                    qw = plsc.load_gather(qr[b], [eidx, cc])
                    kw = plsc.load_gather(kr[b], [eidx, cc])
                    prod = (plsc.bitcast(qw, jnp.bfloat16)
                            * plsc.bitcast(kw, jnp.bfloat16))
                    pe, po = plsc.unpack(prod,
                                         format=plsc.PackFormat.INTERLEAVED)
                    acc = acc + pe + po

                lgb[pl.ds(b * H * SCH + h * SCH + g * 16, 16)] = acc * SCALE
            return carry2

        lax.fori_loop(0, SNGR, group, 0)
        for h in range(H):
            pltpu.async_copy(lgb.at[pl.ds(b * H * SCH + h * SCH, SCH)],
                             out_h.at[pl.ds(h * E + base + c * SCH, SCH)],
                             osem[b])

    for c0 in range(LDEPTH):
        prefetch(jnp.int32(c0), c0)

    def body(i, carry):
        for off in range(LDEPTH):
            c = i * LDEPTH + off
            process(c, off)
            prefetch(c + LDEPTH, off)
        return carry

    lax.fori_loop(0, (SNCH - 1) // LDEPTH, body, 0)
    process(jnp.int32(SNCH - 1), 0)
    for b in range(LDEPTH):
        pltpu.make_async_copy(out_h.at[pl.ds(0, H * SCH)],
                              lgb.at[pl.ds(b * H * SCH, H * SCH)],
                              osem[b]).wait()


def _logits(row4, col4, q, k):
    mesh = plsc.VectorSubcoreMesh(core_axis_name="c", subcore_axis_name="s")
    f = pl.kernel(
        _logits_body,
        out_type=jax.ShapeDtypeStruct((H * E,), jnp.float32),
        mesh=mesh,
        compiler_params=pltpu.CompilerParams(needs_layout_passes=False,
                                             use_tc_tiling_on_sc=False),
        scratch_types=[
            pltpu.VMEM((2, IBLK, SCH), jnp.int32),
            pltpu.VMEM((2, IBLK, SCH), jnp.int32),
        ] + [pltpu.VMEM((SCH, C // 2), jnp.int32)] * 4 + [
            pltpu.VMEM((LDEPTH * H * SCH,), jnp.float32),
        ] + [pltpu.SemaphoreType.DMA] * 4,
    )
    return f(row4, col4, q, k)


def _stats_body(l_ref, m_ref, z_ref, m_s, z_s):
    i = pl.program_id(0)

    @pl.when(i == 0)
    def _():
        m_s[...] = jnp.full((H, 128), -jnp.inf, jnp.float32)
        z_s[...] = jnp.zeros((H, 128), jnp.float32)

    blk = l_ref[...].reshape(H, -1, 128)
    bm = blk.max(axis=1)
    m_old = m_s[...]
    m_new = jnp.maximum(m_old, bm)
    z_s[...] = z_s[...] * jnp.exp(m_old - m_new) + jnp.exp(
        blk - m_new[:, None, :]).sum(axis=1)
    m_s[...] = m_new

    @pl.when(i == pl.num_programs(0) - 1)
    def _():
        mf = m_s[...].max(axis=1, keepdims=True)
        zf = (z_s[...] * jnp.exp(m_s[...] - mf)).sum(axis=1, keepdims=True)
        m_ref[...] = jnp.broadcast_to(mf, (H, 128))
        z_ref[...] = jnp.broadcast_to(zf, (H, 128))


def _stats(logits):
    bl = 16000
    return pl.pallas_call(
        _stats_body,
        grid=(E // bl,),
        in_specs=[pl.BlockSpec((H, bl), lambda i: (0, i))],
        out_specs=[
            pl.BlockSpec((H, 128), lambda i: (0, 0)),
            pl.BlockSpec((H, 128), lambda i: (0, 0)),
        ],
        out_shape=[jax.ShapeDtypeStruct((H, 128), jnp.float32)] * 2,
        scratch_shapes=[
            pltpu.VMEM((H, 128), jnp.float32),
            pltpu.VMEM((H, 128), jnp.float32),
        ],
    )(logits)


def _scatter_body(row4, col4, v_h, lg_h, m_h, z_h, zero_h, out_h,
                  ribuf, cibuf, vr0, vr1, vr2, vr3, lgv, msv, zsv, osh,
                  gsem0, gsem1, gsem2, gsem3,
                  ssem0, ssem1, ssem2, ssem3,
                  lsem0, lsem1, lsem2, lsem3):
    cid = lax.axis_index("c")
    sid = lax.axis_index("s")
    wid = sid * NC + cid
    base = wid * EW
    vr = [vr0, vr1, vr2, vr3]
    gsem = [gsem0, gsem1, gsem2, gsem3]
    ssem = [ssem0, ssem1, ssem2, ssem3]
    lsem = [lsem0, lsem1, lsem2, lsem3]

    pltpu.sync_copy(zero_h.at[pl.ds(sid * RPT, RPT)], osh.at[pl.ds(sid * RPT, RPT)])

    @pl.when(sid == NS - 1)
    def _():
        pltpu.sync_copy(zero_h.at[pl.ds(NS * RPT, N - NS * RPT)],
                        osh.at[pl.ds(NS * RPT, N - NS * RPT)])

    def prefetch(c2, b2):
        @pl.when(c2 <= SNCH - 1)
        def _():
            blk = c2 // IBLK

            @pl.when(c2 % IBLK == 0)
            def _():
                pltpu.sync_copy(row4.at[wid, blk], ribuf.at[blk % 2])
                pltpu.sync_copy(col4.at[wid, blk], cibuf.at[blk % 2])

            @pl.when(c2 >= 4)
            def _():
                pltpu.make_async_copy(v_h.at[pl.ds(0, SCH)], vr[b2],
                                      ssem[b2]).wait()

            for h in range(H):
                pltpu.async_copy(
                    lg_h.at[pl.ds(h * E + base + c2 * SCH, SCH)],
                    lgv.at[pl.ds(b2 * H * SCH + h * SCH, SCH)], lsem[b2])
            pltpu.async_copy(v_h.at[cibuf.at[blk % 2, c2 % IBLK]],
                             vr[b2], gsem[b2])

    prefetch(jnp.int32(0), 0)
    prefetch(jnp.int32(1), 1)

    plsc.subcore_barrier()

    pltpu.sync_copy(m_h, msv)
    pltpu.sync_copy(z_h, zsv)
    ms = [msv[pl.ds(h * 16, 16)] for h in range(H)]
    rzs = [1.0 / zsv[pl.ds(h * 16, 16)] for h in range(H)]

    def process(c, b):
        blk = c // IBLK
        pltpu.make_async_copy(v_h.at[pl.ds(0, SCH)], vr[b], gsem[b]).wait()
        pltpu.make_async_copy(lg_h.at[pl.ds(0, H * SCH)],
                              lgv.at[pl.ds(b * H * SCH, H * SCH)],
                              lsem[b]).wait()
        for h in range(H):
            for j in range(SNGR):
                sl = pl.ds(b * H * SCH + h * SCH + j * 16, 16)
                lgv[sl] = jnp.exp(lgv[sl] - ms[h]) * rzs[h]

        def group(g, carry2):
            for e in range(16):
                ei = jnp.full((16,), g * 16 + e, jnp.int32)
                for h in range(H):
                    w = plsc.load_gather(lgv, [ei + (b * H * SCH + h * SCH)])
                    didx = lax.iota(jnp.int32, 16) + h * DH
                    x = plsc.load_gather(vr[b], [ei, didx])
                    plsc.store_scatter(vr[b], [ei, didx], x * w)
            return carry2

        lax.fori_loop(0, SNGR, group, 0)
        pltpu.async_copy(vr[b], osh.at[ribuf.at[blk % 2, c % IBLK]],
                         ssem[b], add=True)

    def body(i, carry):
        for off in range(4):
            c = i * 4 + off
            process(c, off)
            prefetch(c + 2, (off + 2) % 4)
        return carry

    lax.fori_loop(0, (SNCH - 1) // 4, body, 0)
    process(jnp.int32(SNCH - 1), 0)
    for b in range(4):
        pltpu.make_async_copy(v_h.at[pl.ds(0, SCH)], vr[b], ssem[b]).wait()

    plsc.subcore_barrier()
    pltpu.sync_copy(osh.at[pl.ds(sid * RPT, RPT)],
                    out_h.at[cid, pl.ds(sid * RPT, RPT)])

    @pl.when(sid == NS - 1)
    def _():
        pltpu.sync_copy(osh.at[pl.ds(NS * RPT, N - NS * RPT)],
                        out_h.at[cid, pl.ds(NS * RPT, N - NS * RPT)])


def _scatter(row4, col4, v, logits, m, z, zeros):
    mesh = plsc.VectorSubcoreMesh(core_axis_name="c", subcore_axis_name="s")
    f = pl.kernel(
        _scatter_body,
        out_type=jax.ShapeDtypeStruct((NC, N, C), jnp.float32),
        mesh=mesh,
        compiler_params=pltpu.CompilerParams(needs_layout_passes=False),
        scratch_types=[
            pltpu.VMEM((2, IBLK, SCH), jnp.int32),
            pltpu.VMEM((2, IBLK, SCH), jnp.int32),
            pltpu.VMEM((SCH, C), jnp.float32),
            pltpu.VMEM((SCH, C), jnp.float32),
            pltpu.VMEM((SCH, C), jnp.float32),
            pltpu.VMEM((SCH, C), jnp.float32),
            pltpu.VMEM((4 * H * SCH,), jnp.float32),
            pltpu.VMEM((H * 16,), jnp.float32),
            pltpu.VMEM((H * 16,), jnp.float32),
            pltpu.VMEM_SHARED((N, C), jnp.float32),
        ] + [pltpu.SemaphoreType.DMA] * 12,
    )
    return f(row4, col4, v, logits, m, z, zeros)


def _out_body(p0_ref, p1_ref, w_ref, b_ref, o_ref):
    o_ref[...] = jnp.dot(p0_ref[0] + p1_ref[0], w_ref[...],
                         preferred_element_type=jnp.float32) + b_ref[...]


def _outproj(partials, wo_t, bo):
    bn = 1000
    return pl.pallas_call(
        _out_body,
        grid=(N // bn,),
        in_specs=[
            pl.BlockSpec((1, bn, C), lambda i: (0, i, 0)),
            pl.BlockSpec((1, bn, C), lambda i: (1, i, 0)),
            pl.BlockSpec((C, C), lambda i: (0, 0)),
            pl.BlockSpec((1, C), lambda i: (0, 0)),
        ],
        out_specs=pl.BlockSpec((bn, C), lambda i: (i, 0)),
        out_shape=jax.ShapeDtypeStruct((N, C), jnp.float32),
    )(partials, partials, wo_t, bo)


def kernel(features, edge_index, Wq, bq, Wk, bk, Wv, bv, Wo, bo):
    row4 = edge_index[0].astype(jnp.int32).reshape(NW, NBLK, IBLK, SCH)
    col4 = edge_index[1].astype(jnp.int32).reshape(NW, NBLK, IBLK, SCH)
    wcat = jnp.concatenate([Wq.T, Wk.T, Wv.T], axis=1)
    bcat = jnp.concatenate([bq, bk, bv]).reshape(1, 3 * C)
    q, k, v = _qkv(features, wcat, bcat)
    qpair = lax.bitcast_convert_type(q.reshape(N, C // 2, 2), jnp.int32)
    kpair = lax.bitcast_convert_type(k.reshape(N, C // 2, 2), jnp.int32)
    logits = _logits(row4, col4, qpair, kpair)
    m, z = _stats(logits.reshape(H, E))
    m16 = lax.slice(m, (0, 0), (H, 16)).reshape(H * 16)
    z16 = lax.slice(z, (0, 0), (H, 16)).reshape(H * 16)
    zeros = jnp.zeros((N, C), jnp.float32)
    partials = _scatter(row4, col4, v, logits, m16, z16, zeros)
    return _outproj(partials, Wo.T, bo.reshape(1, C))


# speedup vs baseline: 1.5802x; 1.1280x over previous
import math

import jax
import jax.numpy as jnp
from jax import lax
from jax.experimental import pallas as pl
from jax.experimental.pallas import tpu as pltpu
from jax.experimental.pallas import tpu_sc as plsc

N = 10000
E = 320000
C = 128
H = 8
DH = 16
SCALE = 1.0 / math.sqrt(DH)

NC = 2
NS = 16
NW = NC * NS
EW = E // NW

SCH = 80
SNCH = EW // SCH
SNGR = SCH // 16
IBLK = 5
NBLK = SNCH // IBLK

RPT = 624


def _qkv_body(f_ref, w_ref, b_ref, oq_ref, ok_ref, ov_ref):
    acc = jnp.dot(f_ref[...], w_ref[...], preferred_element_type=jnp.float32)
    acc = acc + b_ref[...]
    oq_ref[...] = acc[:, 0:C].astype(jnp.bfloat16)
    ok_ref[...] = acc[:, C:2 * C].astype(jnp.bfloat16)
    ov_ref[...] = acc[:, 2 * C:3 * C].astype(jnp.bfloat16)


def _qkv(features, wcat, bcat):
    bn = 1000
    return pl.pallas_call(
        _qkv_body,
        grid=(N // bn,),
        in_specs=[
            pl.BlockSpec((bn, C), lambda i: (i, 0)),
            pl.BlockSpec((C, 3 * C), lambda i: (0, 0)),
            pl.BlockSpec((1, 3 * C), lambda i: (0, 0)),
        ],
        out_specs=[
            pl.BlockSpec((bn, C), lambda i: (i, 0)),
            pl.BlockSpec((bn, C), lambda i: (i, 0)),
            pl.BlockSpec((bn, C), lambda i: (i, 0)),
        ],
        out_shape=[jax.ShapeDtypeStruct((N, C), jnp.bfloat16)] * 3,
    )(features, wcat, bcat)


LDEPTH = 2


def _logits_body(row4, col4, q_h, k_h, out_h,
                 ribuf, cibuf,
                 qr0, qr1, kr0, kr1, lgb,
                 gsem0, gsem1,
                 osem0, osem1):
    cid = lax.axis_index("c")
    sid = lax.axis_index("s")
    wid = sid * NC + cid
    base = wid * EW
    qr = [qr0, qr1]
    kr = [kr0, kr1]
    gsem = [gsem0, gsem1]
    osem = [osem0, osem1]

    def prefetch(c2, b2):
        @pl.when(c2 <= SNCH - 1)
        def _():
            blk = c2 // IBLK

            @pl.when(c2 % IBLK == 0)
            def _():
                pltpu.sync_copy(row4.at[wid, blk], ribuf.at[blk % 2])
                pltpu.sync_copy(col4.at[wid, blk], cibuf.at[blk % 2])

            pltpu.async_copy(q_h.at[ribuf.at[blk % 2, c2 % IBLK]],
                             qr[b2], gsem[b2])
            pltpu.async_copy(k_h.at[cibuf.at[blk % 2, c2 % IBLK]],
                             kr[b2], gsem[b2])

    def process(c, b):
        pltpu.make_async_copy(q_h.at[pl.ds(0, SCH)], qr[b], gsem[b]).wait()
        pltpu.make_async_copy(q_h.at[pl.ds(0, SCH)], kr[b], gsem[b]).wait()

        @pl.when(c >= LDEPTH)
        def _():
            pltpu.make_async_copy(out_h.at[pl.ds(0, H * SCH)],
                                  lgb.at[pl.ds(b * H * SCH, H * SCH)],
                                  osem[b]).wait()

        def group(g, carry2):
            eidx = lax.iota(jnp.int32, 16) + g * 16
            for h in range(H):
                acc = jnp.zeros((16,), jnp.float32)
                for w in range(DH // 2):
                    cc = jnp.full((16,), h * (DH // 2) + w, jnp.int32)
                    qw = plsc.load_gather(qr[b], [eidx, cc])
                    kw = plsc.load_gather(kr[b], [eidx, cc])
                    prod = (plsc.bitcast(qw, jnp.bfloat16)
                            * plsc.bitcast(kw, jnp.bfloat16))
                    pe, po = plsc.unpack(prod,
                                         format=plsc.PackFormat.INTERLEAVED)
                    acc = acc + pe + po

                lgb[pl.ds(b * H * SCH + h * SCH + g * 16, 16)] = acc * SCALE
            return carry2

        lax.fori_loop(0, SNGR, group, 0)
        for h in range(H):
            pltpu.async_copy(lgb.at[pl.ds(b * H * SCH + h * SCH, SCH)],
                             out_h.at[pl.ds(h * E + base + c * SCH, SCH)],
                             osem[b])

    for c0 in range(LDEPTH):
        prefetch(jnp.int32(c0), c0)

    def body(i, carry):
        for off in range(LDEPTH):
            c = i * LDEPTH + off
            process(c, off)
            prefetch(c + LDEPTH, off)
        return carry

    lax.fori_loop(0, (SNCH - 1) // LDEPTH, body, 0)
    process(jnp.int32(SNCH - 1), 0)
    for b in range(LDEPTH):
        pltpu.make_async_copy(out_h.at[pl.ds(0, H * SCH)],
                              lgb.at[pl.ds(b * H * SCH, H * SCH)],
                              osem[b]).wait()


def _logits(row4, col4, q, k):
    mesh = plsc.VectorSubcoreMesh(core_axis_name="c", subcore_axis_name="s")
    f = pl.kernel(
        _logits_body,
        out_type=jax.ShapeDtypeStruct((H * E,), jnp.float32),
        mesh=mesh,
        compiler_params=pltpu.CompilerParams(needs_layout_passes=False,
                                             use_tc_tiling_on_sc=False),
        scratch_types=[
            pltpu.VMEM((2, IBLK, SCH), jnp.int32),
            pltpu.VMEM((2, IBLK, SCH), jnp.int32),
        ] + [pltpu.VMEM((SCH, C // 2), jnp.int32)] * 4 + [
            pltpu.VMEM((LDEPTH * H * SCH,), jnp.float32),
        ] + [pltpu.SemaphoreType.DMA] * 4,
    )
    return f(row4, col4, q, k)


def _stats_body(l_ref, m_ref, z_ref, m_s, z_s):
    i = pl.program_id(0)

    @pl.when(i == 0)
    def _():
        m_s[...] = jnp.full((H, 128), -jnp.inf, jnp.float32)
        z_s[...] = jnp.zeros((H, 128), jnp.float32)

    blk = l_ref[...].reshape(H, -1, 128)
    bm = blk.max(axis=1)
    m_old = m_s[...]
    m_new = jnp.maximum(m_old, bm)
    z_s[...] = z_s[...] * jnp.exp(m_old - m_new) + jnp.exp(
        blk - m_new[:, None, :]).sum(axis=1)
    m_s[...] = m_new

    @pl.when(i == pl.num_programs(0) - 1)
    def _():
        mf = m_s[...].max(axis=1, keepdims=True)
        zf = (z_s[...] * jnp.exp(m_s[...] - mf)).sum(axis=1, keepdims=True)
        m_ref[...] = jnp.broadcast_to(mf, (H, 128))
        z_ref[...] = jnp.broadcast_to(zf, (H, 128))


def _stats(logits):
    bl = 16000
    return pl.pallas_call(
        _stats_body,
        grid=(E // bl,),
        in_specs=[pl.BlockSpec((H, bl), lambda i: (0, i))],
        out_specs=[
            pl.BlockSpec((H, 128), lambda i: (0, 0)),
            pl.BlockSpec((H, 128), lambda i: (0, 0)),
        ],
        out_shape=[jax.ShapeDtypeStruct((H, 128), jnp.float32)] * 2,
        scratch_shapes=[
            pltpu.VMEM((H, 128), jnp.float32),
            pltpu.VMEM((H, 128), jnp.float32),
        ],
    )(logits)


def _scatter_body(row4, col4, v_h, lg_h, m_h, z_h, zero_h, out_h,
                  ribuf, cibuf, vr0, vr1, vr2, vr3, vrf0, vrf1,
                  lgv, msv, zsv, osh,
                  gsem0, gsem1, gsem2, gsem3,
                  ssem0, ssem1,
                  lsem0, lsem1, lsem2, lsem3):
    cid = lax.axis_index("c")
    sid = lax.axis_index("s")
    wid = sid * NC + cid
    base = wid * EW
    vr = [vr0, vr1, vr2, vr3]
    vrf = [vrf0, vrf1]
    gsem = [gsem0, gsem1, gsem2, gsem3]
    ssem = [ssem0, ssem1]
    lsem = [lsem0, lsem1, lsem2, lsem3]

    pltpu.sync_copy(zero_h.at[pl.ds(sid * RPT, RPT)], osh.at[pl.ds(sid * RPT, RPT)])

    @pl.when(sid == NS - 1)
    def _():
        pltpu.sync_copy(zero_h.at[pl.ds(NS * RPT, N - NS * RPT)],
                        osh.at[pl.ds(NS * RPT, N - NS * RPT)])

    def prefetch(c2, b2):
        @pl.when(c2 <= SNCH - 1)
        def _():
            blk = c2 // IBLK

            @pl.when(c2 % IBLK == 0)
            def _():
                pltpu.sync_copy(row4.at[wid, blk], ribuf.at[blk % 2])
                pltpu.sync_copy(col4.at[wid, blk], cibuf.at[blk % 2])

            for h in range(H):
                pltpu.async_copy(
                    lg_h.at[pl.ds(h * E + base + c2 * SCH, SCH)],
                    lgv.at[pl.ds(b2 * H * SCH + h * SCH, SCH)], lsem[b2])
            pltpu.async_copy(v_h.at[cibuf.at[blk % 2, c2 % IBLK]],
                             vr[b2], gsem[b2])

    prefetch(jnp.int32(0), 0)
    prefetch(jnp.int32(1), 1)

    plsc.subcore_barrier()

    pltpu.sync_copy(m_h, msv)
    pltpu.sync_copy(z_h, zsv)
    ms = [msv[pl.ds(h * 16, 16)] for h in range(H)]
    rzs = [1.0 / zsv[pl.ds(h * 16, 16)] for h in range(H)]

    selv = jnp.where(lax.iota(jnp.int32, 16) >= 8, SCH, 0)
    iot2 = lax.iota(jnp.int32, 16) * 2

    def process(c, b, pb):
        blk = c // IBLK
        pltpu.make_async_copy(v_h.at[pl.ds(0, SCH)], vr[b], gsem[b]).wait()
        pltpu.make_async_copy(lg_h.at[pl.ds(0, H * SCH)],
                              lgv.at[pl.ds(b * H * SCH, H * SCH)],
                              lsem[b]).wait()
        @pl.when(c >= 2)
        def _():
            pltpu.make_async_copy(zero_h.at[pl.ds(0, SCH)], vrf[pb],
                                  ssem[pb]).wait()

        for h in range(H):
            for j in range(SNGR):
                sl = pl.ds(b * H * SCH + h * SCH + j * 16, 16)
                lgv[sl] = jnp.exp(lgv[sl] - ms[h]) * rzs[h]

        def group(g, carry2):
            for e in range(16):
                ei = jnp.full((16,), g * 16 + e, jnp.int32)
                for hp in range(H // 2):
                    words = plsc.load_gather(
                        vr[b], [ei, lax.iota(jnp.int32, 16) + hp * 16])
                    bf = plsc.bitcast(words, jnp.bfloat16)
                    pe, po = plsc.unpack(bf,
                                         format=plsc.PackFormat.INTERLEAVED)
                    wmix = plsc.load_gather(
                        lgv, [ei + (b * H * SCH + 2 * hp * SCH) + selv])
                    plsc.store_scatter(vrf[pb], [ei, iot2 + hp * 32],
                                       pe * wmix)
                    plsc.store_scatter(vrf[pb], [ei, iot2 + (hp * 32 + 1)],
                                       po * wmix)
            return carry2

        lax.fori_loop(0, SNGR, group, 0)
        pltpu.async_copy(vrf[pb], osh.at[ribuf.at[blk % 2, c % IBLK]],
                         ssem[pb], add=True)

    def body(i, carry):
        for off in range(4):
            c = i * 4 + off
            process(c, off, off % 2)
            prefetch(c + 2, (off + 2) % 4)
        return carry

    lax.fori_loop(0, (SNCH - 1) // 4, body, 0)
    process(jnp.int32(SNCH - 1), 0, 0)
    for pb in range(2):
        pltpu.make_async_copy(zero_h.at[pl.ds(0, SCH)], vrf[pb],
                              ssem[pb]).wait()

    plsc.subcore_barrier()
    pltpu.sync_copy(osh.at[pl.ds(sid * RPT, RPT)],
                    out_h.at[cid, pl.ds(sid * RPT, RPT)])

    @pl.when(sid == NS - 1)
    def _():
        pltpu.sync_copy(osh.at[pl.ds(NS * RPT, N - NS * RPT)],
                        out_h.at[cid, pl.ds(NS * RPT, N - NS * RPT)])


def _scatter(row4, col4, v, logits, m, z, zeros):
    mesh = plsc.VectorSubcoreMesh(core_axis_name="c", subcore_axis_name="s")
    f = pl.kernel(
        _scatter_body,
        out_type=jax.ShapeDtypeStruct((NC, N, C), jnp.float32),
        mesh=mesh,
        compiler_params=pltpu.CompilerParams(needs_layout_passes=False,
                                             use_tc_tiling_on_sc=False),
        scratch_types=[
            pltpu.VMEM((2, IBLK, SCH), jnp.int32),
            pltpu.VMEM((2, IBLK, SCH), jnp.int32),
        ] + [pltpu.VMEM((SCH, C // 2), jnp.int32)] * 4 + [
            pltpu.VMEM((SCH, C), jnp.float32),
            pltpu.VMEM((SCH, C), jnp.float32),
            pltpu.VMEM((4 * H * SCH,), jnp.float32),
            pltpu.VMEM((H * 16,), jnp.float32),
            pltpu.VMEM((H * 16,), jnp.float32),
            pltpu.VMEM_SHARED((N, C), jnp.float32),
        ] + [pltpu.SemaphoreType.DMA] * 10,
    )
    return f(row4, col4, v, logits, m, z, zeros)


def _out_body(p0_ref, p1_ref, w_ref, b_ref, o_ref):
    o_ref[...] = jnp.dot(p0_ref[0] + p1_ref[0], w_ref[...],
                         preferred_element_type=jnp.float32) + b_ref[...]


def _outproj(partials, wo_t, bo):
    bn = 1000
    return pl.pallas_call(
        _out_body,
        grid=(N // bn,),
        in_specs=[
            pl.BlockSpec((1, bn, C), lambda i: (0, i, 0)),
            pl.BlockSpec((1, bn, C), lambda i: (1, i, 0)),
            pl.BlockSpec((C, C), lambda i: (0, 0)),
            pl.BlockSpec((1, C), lambda i: (0, 0)),
        ],
        out_specs=pl.BlockSpec((bn, C), lambda i: (i, 0)),
        out_shape=jax.ShapeDtypeStruct((N, C), jnp.float32),
    )(partials, partials, wo_t, bo)


def kernel(features, edge_index, Wq, bq, Wk, bk, Wv, bv, Wo, bo):
    row4 = edge_index[0].astype(jnp.int32).reshape(NW, NBLK, IBLK, SCH)
    col4 = edge_index[1].astype(jnp.int32).reshape(NW, NBLK, IBLK, SCH)
    wcat = jnp.concatenate([Wq.T, Wk.T, Wv.T], axis=1)
    bcat = jnp.concatenate([bq, bk, bv]).reshape(1, 3 * C)
    q, k, v = _qkv(features, wcat, bcat)
    qpair = lax.bitcast_convert_type(q.reshape(N, C // 2, 2), jnp.int32)
    kpair = lax.bitcast_convert_type(k.reshape(N, C // 2, 2), jnp.int32)
    vpair = lax.bitcast_convert_type(v.reshape(N, C // 2, 2), jnp.int32)
    logits = _logits(row4, col4, qpair, kpair)
    m, z = _stats(logits.reshape(H, E))
    m16 = lax.slice(m, (0, 0), (H, 16)).reshape(H * 16)
    z16 = lax.slice(z, (0, 0), (H, 16)).reshape(H * 16)
    zeros = jnp.zeros((N, C), jnp.float32)
    partials = _scatter(row4, col4, vpair, logits, m16, z16, zeros)
    return _outproj(partials, Wo.T, bo.reshape(1, C))


# speedup vs baseline: 1.6055x; 1.0160x over previous
import math

import jax
import jax.numpy as jnp
from jax import lax
from jax.experimental import pallas as pl
from jax.experimental.pallas import tpu as pltpu
from jax.experimental.pallas import tpu_sc as plsc

N = 10000
E = 320000
C = 128
H = 8
DH = 16
SCALE = 1.0 / math.sqrt(DH)

NC = 2
NS = 16
NW = NC * NS
EW = E // NW

SCH = 80
SNCH = EW // SCH
SNGR = SCH // 16
IBLK = 5
NBLK = SNCH // IBLK

RPT = 624


def _qkv_body(f_ref, w_ref, b_ref, oq_ref, ok_ref, ov_ref):
    acc = jnp.dot(f_ref[...], w_ref[...], preferred_element_type=jnp.float32)
    acc = acc + b_ref[...]
    oq_ref[...] = acc[:, 0:C].astype(jnp.bfloat16)
    ok_ref[...] = acc[:, C:2 * C].astype(jnp.bfloat16)
    ov_ref[...] = acc[:, 2 * C:3 * C].astype(jnp.bfloat16)


def _qkv(features, wcat, bcat):
    bn = 1000
    return pl.pallas_call(
        _qkv_body,
        grid=(N // bn,),
        in_specs=[
            pl.BlockSpec((bn, C), lambda i: (i, 0)),
            pl.BlockSpec((C, 3 * C), lambda i: (0, 0)),
            pl.BlockSpec((1, 3 * C), lambda i: (0, 0)),
        ],
        out_specs=[
            pl.BlockSpec((bn, C), lambda i: (i, 0)),
            pl.BlockSpec((bn, C), lambda i: (i, 0)),
            pl.BlockSpec((bn, C), lambda i: (i, 0)),
        ],
        out_shape=[jax.ShapeDtypeStruct((N, C), jnp.bfloat16)] * 3,
    )(features, wcat, bcat)


LSCH = 400
LNCH = EW // LSCH
LNGR = LSCH // 16
LNBLK = LNCH // IBLK
LDEPTH = 2


def _logits_body(row4, col4, q_h, k_h, out_h,
                 ribuf, cibuf,
                 qr0, qr1, kr0, kr1, lgb,
                 gsem0, gsem1,
                 osem0, osem1):
    cid = lax.axis_index("c")
    sid = lax.axis_index("s")
    wid = sid * NC + cid
    base = wid * EW
    qr = [qr0, qr1]
    kr = [kr0, kr1]
    gsem = [gsem0, gsem1]
    osem = [osem0, osem1]

    def prefetch(c2, b2):
        @pl.when(c2 <= LNCH - 1)
        def _():
            blk = c2 // IBLK

            @pl.when(c2 % IBLK == 0)
            def _():
                pltpu.sync_copy(row4.at[wid, blk], ribuf.at[blk % 2])
                pltpu.sync_copy(col4.at[wid, blk], cibuf.at[blk % 2])

            pltpu.async_copy(q_h.at[ribuf.at[blk % 2, c2 % IBLK]],
                             qr[b2], gsem[b2])
            pltpu.async_copy(k_h.at[cibuf.at[blk % 2, c2 % IBLK]],
                             kr[b2], gsem[b2])

    def process(c, b):
        pltpu.make_async_copy(q_h.at[pl.ds(0, LSCH)], qr[b], gsem[b]).wait()
        pltpu.make_async_copy(q_h.at[pl.ds(0, LSCH)], kr[b], gsem[b]).wait()

        @pl.when(c >= LDEPTH)
        def _():
            pltpu.make_async_copy(out_h.at[pl.ds(0, H * LSCH)],
                                  lgb.at[pl.ds(b * H * LSCH, H * LSCH)],
                                  osem[b]).wait()

        def group(g, carry2):
            eidx = lax.iota(jnp.int32, 16) + g * 16
            for h in range(H):
                acc = jnp.zeros((16,), jnp.float32)
                for w in range(DH // 2):
                    cc = jnp.full((16,), h * (DH // 2) + w, jnp.int32)
                    qw = plsc.load_gather(qr[b], [eidx, cc])
                    kw = plsc.load_gather(kr[b], [eidx, cc])
                    prod = (plsc.bitcast(qw, jnp.bfloat16)
                            * plsc.bitcast(kw, jnp.bfloat16))
                    pe, po = plsc.unpack(prod,
                                         format=plsc.PackFormat.INTERLEAVED)
                    acc = acc + pe + po

                lgb[pl.ds(b * H * LSCH + h * LSCH + g * 16, 16)] = acc * SCALE
            return carry2

        lax.fori_loop(0, LNGR, group, 0)
        for h in range(H):
            pltpu.async_copy(lgb.at[pl.ds(b * H * LSCH + h * LSCH, LSCH)],
                             out_h.at[pl.ds(h * E + base + c * LSCH, LSCH)],
                             osem[b])

    for c0 in range(LDEPTH):
        prefetch(jnp.int32(c0), c0)

    def body(i, carry):
        for off in range(LDEPTH):
            c = i * LDEPTH + off
            process(c, off)
            prefetch(c + LDEPTH, off)
        return carry

    lax.fori_loop(0, (LNCH - 1) // LDEPTH, body, 0)
    process(jnp.int32(LNCH - 1), 0)
    for b in range(LDEPTH):
        pltpu.make_async_copy(out_h.at[pl.ds(0, H * LSCH)],
                              lgb.at[pl.ds(b * H * LSCH, H * LSCH)],
                              osem[b]).wait()


def _logits(row4, col4, q, k):
    mesh = plsc.VectorSubcoreMesh(core_axis_name="c", subcore_axis_name="s")
    f = pl.kernel(
        _logits_body,
        out_type=jax.ShapeDtypeStruct((H * E,), jnp.float32),
        mesh=mesh,
        compiler_params=pltpu.CompilerParams(needs_layout_passes=False,
                                             use_tc_tiling_on_sc=False),
        scratch_types=[
            pltpu.VMEM((2, IBLK, LSCH), jnp.int32),
            pltpu.VMEM((2, IBLK, LSCH), jnp.int32),
        ] + [pltpu.VMEM((LSCH, C // 2), jnp.int32)] * 4 + [
            pltpu.VMEM((LDEPTH * H * LSCH,), jnp.float32),
        ] + [pltpu.SemaphoreType.DMA] * 4,
    )
    return f(row4, col4, q, k)


def _stats_body(l_ref, m_ref, z_ref, m_s, z_s):
    i = pl.program_id(0)

    @pl.when(i == 0)
    def _():
        m_s[...] = jnp.full((H, 128), -jnp.inf, jnp.float32)
        z_s[...] = jnp.zeros((H, 128), jnp.float32)

    blk = l_ref[...].reshape(H, -1, 128)
    bm = blk.max(axis=1)
    m_old = m_s[...]
    m_new = jnp.maximum(m_old, bm)
    z_s[...] = z_s[...] * jnp.exp(m_old - m_new) + jnp.exp(
        blk - m_new[:, None, :]).sum(axis=1)
    m_s[...] = m_new

    @pl.when(i == pl.num_programs(0) - 1)
    def _():
        mf = m_s[...].max(axis=1, keepdims=True)
        zf = (z_s[...] * jnp.exp(m_s[...] - mf)).sum(axis=1, keepdims=True)
        m_ref[...] = jnp.broadcast_to(mf, (H, 128))
        z_ref[...] = jnp.broadcast_to(zf, (H, 128))


def _stats(logits):
    bl = 16000
    return pl.pallas_call(
        _stats_body,
        grid=(E // bl,),
        in_specs=[pl.BlockSpec((H, bl), lambda i: (0, i))],
        out_specs=[
            pl.BlockSpec((H, 128), lambda i: (0, 0)),
            pl.BlockSpec((H, 128), lambda i: (0, 0)),
        ],
        out_shape=[jax.ShapeDtypeStruct((H, 128), jnp.float32)] * 2,
        scratch_shapes=[
            pltpu.VMEM((H, 128), jnp.float32),
            pltpu.VMEM((H, 128), jnp.float32),
        ],
    )(logits)


def _scatter_body(row4, col4, v_h, lg_h, m_h, z_h, zero_h, out_h,
                  ribuf, cibuf, vr0, vr1, vr2, vr3, vrf0, vrf1,
                  lgv, msv, zsv, osh,
                  gsem0, gsem1, gsem2, gsem3,
                  ssem0, ssem1,
                  lsem0, lsem1, lsem2, lsem3):
    cid = lax.axis_index("c")
    sid = lax.axis_index("s")
    wid = sid * NC + cid
    base = wid * EW
    vr = [vr0, vr1, vr2, vr3]
    vrf = [vrf0, vrf1]
    gsem = [gsem0, gsem1, gsem2, gsem3]
    ssem = [ssem0, ssem1]
    lsem = [lsem0, lsem1, lsem2, lsem3]

    pltpu.sync_copy(zero_h.at[pl.ds(sid * RPT, RPT)], osh.at[pl.ds(sid * RPT, RPT)])

    @pl.when(sid == NS - 1)
    def _():
        pltpu.sync_copy(zero_h.at[pl.ds(NS * RPT, N - NS * RPT)],
                        osh.at[pl.ds(NS * RPT, N - NS * RPT)])

    def prefetch(c2, b2):
        @pl.when(c2 <= SNCH - 1)
        def _():
            blk = c2 // IBLK

            @pl.when(c2 % IBLK == 0)
            def _():
                pltpu.sync_copy(row4.at[wid, blk], ribuf.at[blk % 2])
                pltpu.sync_copy(col4.at[wid, blk], cibuf.at[blk % 2])

            for h in range(H):
                pltpu.async_copy(
                    lg_h.at[pl.ds(h * E + base + c2 * SCH, SCH)],
                    lgv.at[pl.ds(b2 * H * SCH + h * SCH, SCH)], lsem[b2])
            pltpu.async_copy(v_h.at[cibuf.at[blk % 2, c2 % IBLK]],
                             vr[b2], gsem[b2])

    prefetch(jnp.int32(0), 0)
    prefetch(jnp.int32(1), 1)

    plsc.subcore_barrier()

    pltpu.sync_copy(m_h, msv)
    pltpu.sync_copy(z_h, zsv)
    ms = [msv[pl.ds(h * 16, 16)] for h in range(H)]
    rzs = [1.0 / zsv[pl.ds(h * 16, 16)] for h in range(H)]

    selv = jnp.where(lax.iota(jnp.int32, 16) >= 8, SCH, 0)
    iot2 = lax.iota(jnp.int32, 16) * 2

    def process(c, b, pb):
        blk = c // IBLK
        pltpu.make_async_copy(v_h.at[pl.ds(0, SCH)], vr[b], gsem[b]).wait()
        pltpu.make_async_copy(lg_h.at[pl.ds(0, H * SCH)],
                              lgv.at[pl.ds(b * H * SCH, H * SCH)],
                              lsem[b]).wait()
        @pl.when(c >= 2)
        def _():
            pltpu.make_async_copy(zero_h.at[pl.ds(0, SCH)], vrf[pb],
                                  ssem[pb]).wait()

        for h in range(H):
            for j in range(SNGR):
                sl = pl.ds(b * H * SCH + h * SCH + j * 16, 16)
                lgv[sl] = jnp.exp(lgv[sl] - ms[h]) * rzs[h]

        def group(g, carry2):
            for e in range(16):
                ei = jnp.full((16,), g * 16 + e, jnp.int32)
                for hp in range(H // 2):
                    words = plsc.load_gather(
                        vr[b], [ei, lax.iota(jnp.int32, 16) + hp * 16])
                    bf = plsc.bitcast(words, jnp.bfloat16)
                    pe, po = plsc.unpack(bf,
                                         format=plsc.PackFormat.INTERLEAVED)
                    wmix = plsc.load_gather(
                        lgv, [ei + (b * H * SCH + 2 * hp * SCH) + selv])
                    plsc.store_scatter(vrf[pb], [ei, iot2 + hp * 32],
                                       pe * wmix)
                    plsc.store_scatter(vrf[pb], [ei, iot2 + (hp * 32 + 1)],
                                       po * wmix)
            return carry2

        lax.fori_loop(0, SNGR, group, 0)
        pltpu.async_copy(vrf[pb], osh.at[ribuf.at[blk % 2, c % IBLK]],
                         ssem[pb], add=True)

    def body(i, carry):
        for off in range(4):
            c = i * 4 + off
            process(c, off, off % 2)
            prefetch(c + 2, (off + 2) % 4)
        return carry

    lax.fori_loop(0, (SNCH - 1) // 4, body, 0)
    process(jnp.int32(SNCH - 1), 0, 0)
    for pb in range(2):
        pltpu.make_async_copy(zero_h.at[pl.ds(0, SCH)], vrf[pb],
                              ssem[pb]).wait()

    plsc.subcore_barrier()
    pltpu.sync_copy(osh.at[pl.ds(sid * RPT, RPT)],
                    out_h.at[cid, pl.ds(sid * RPT, RPT)])

    @pl.when(sid == NS - 1)
    def _():
        pltpu.sync_copy(osh.at[pl.ds(NS * RPT, N - NS * RPT)],
                        out_h.at[cid, pl.ds(NS * RPT, N - NS * RPT)])


def _scatter(row4, col4, v, logits, m, z, zeros):
    mesh = plsc.VectorSubcoreMesh(core_axis_name="c", subcore_axis_name="s")
    f = pl.kernel(
        _scatter_body,
        out_type=jax.ShapeDtypeStruct((NC, N, C), jnp.float32),
        mesh=mesh,
        compiler_params=pltpu.CompilerParams(needs_layout_passes=False,
                                             use_tc_tiling_on_sc=False),
        scratch_types=[
            pltpu.VMEM((2, IBLK, SCH), jnp.int32),
            pltpu.VMEM((2, IBLK, SCH), jnp.int32),
        ] + [pltpu.VMEM((SCH, C // 2), jnp.int32)] * 4 + [
            pltpu.VMEM((SCH, C), jnp.float32),
            pltpu.VMEM((SCH, C), jnp.float32),
            pltpu.VMEM((4 * H * SCH,), jnp.float32),
            pltpu.VMEM((H * 16,), jnp.float32),
            pltpu.VMEM((H * 16,), jnp.float32),
            pltpu.VMEM_SHARED((N, C), jnp.float32),
        ] + [pltpu.SemaphoreType.DMA] * 10,
    )
    return f(row4, col4, v, logits, m, z, zeros)


def _out_body(p0_ref, p1_ref, w_ref, b_ref, o_ref):
    o_ref[...] = jnp.dot(p0_ref[0] + p1_ref[0], w_ref[...],
                         preferred_element_type=jnp.float32) + b_ref[...]


def _outproj(partials, wo_t, bo):
    bn = 1000
    return pl.pallas_call(
        _out_body,
        grid=(N // bn,),
        in_specs=[
            pl.BlockSpec((1, bn, C), lambda i: (0, i, 0)),
            pl.BlockSpec((1, bn, C), lambda i: (1, i, 0)),
            pl.BlockSpec((C, C), lambda i: (0, 0)),
            pl.BlockSpec((1, C), lambda i: (0, 0)),
        ],
        out_specs=pl.BlockSpec((bn, C), lambda i: (i, 0)),
        out_shape=jax.ShapeDtypeStruct((N, C), jnp.float32),
    )(partials, partials, wo_t, bo)


def kernel(features, edge_index, Wq, bq, Wk, bk, Wv, bv, Wo, bo):
    row = edge_index[0].astype(jnp.int32)
    col = edge_index[1].astype(jnp.int32)
    row4 = row.reshape(NW, NBLK, IBLK, SCH)
    col4 = col.reshape(NW, NBLK, IBLK, SCH)
    row4l = row.reshape(NW, LNBLK, IBLK, LSCH)
    col4l = col.reshape(NW, LNBLK, IBLK, LSCH)
    wcat = jnp.concatenate([Wq.T, Wk.T, Wv.T], axis=1)
    bcat = jnp.concatenate([bq, bk, bv]).reshape(1, 3 * C)
    q, k, v = _qkv(features, wcat, bcat)
    qpair = lax.bitcast_convert_type(q.reshape(N, C // 2, 2), jnp.int32)
    kpair = lax.bitcast_convert_type(k.reshape(N, C // 2, 2), jnp.int32)
    vpair = lax.bitcast_convert_type(v.reshape(N, C // 2, 2), jnp.int32)
    logits = _logits(row4l, col4l, qpair, kpair)
    m, z = _stats(logits.reshape(H, E))
    m16 = lax.slice(m, (0, 0), (H, 16)).reshape(H * 16)
    z16 = lax.slice(z, (0, 0), (H, 16)).reshape(H * 16)
    zeros = jnp.zeros((N, C), jnp.float32)
    partials = _scatter(row4, col4, vpair, logits, m16, z16, zeros)
    return _outproj(partials, Wo.T, bo.reshape(1, C))
